# baseline (speedup 1.0000x reference)
import functools

import jax
import jax.numpy as jnp
from jax import lax
from jax.experimental import pallas as pl

_ROWS = 512
_VOCAB = 1000
_EMB = 64


def _argmax_embed_kernel(x_ref, dict_ref, out_ref):
    xb = x_ref[...]
    iota = lax.broadcasted_iota(jnp.int32, xb.shape, 1)
    m = jnp.max(xb, axis=-1, keepdims=True)
    eq = xb == m
    idx = jnp.min(jnp.where(eq, iota, _VOCAB), axis=-1, keepdims=True)
    onehot = (iota == idx).astype(jnp.float32)
    out_ref[...] = jnp.dot(onehot, dict_ref[...],
                           preferred_element_type=jnp.float32)


@jax.jit
def kernel(x, dictionary_weight):
    b, s, v = x.shape
    n = b * s
    x2 = x.reshape(n, v)
    grid = (n // _ROWS,)
    out = pl.pallas_call(
        _argmax_embed_kernel,
        grid=grid,
        in_specs=[
            pl.BlockSpec((_ROWS, v), lambda i: (i, 0)),
            pl.BlockSpec((_VOCAB, _EMB), lambda i: (0, 0)),
        ],
        out_specs=pl.BlockSpec((_ROWS, _EMB), lambda i: (i, 0)),
        out_shape=jax.ShapeDtypeStruct((n, _EMB), jnp.float32),
    )(x2, dictionary_weight)
    return out.reshape(b, s, _EMB)


# speedup vs baseline: 1.0006x; 1.0006x over previous
import functools

import jax
import jax.numpy as jnp
from jax import lax
from jax.experimental import pallas as pl
from jax.experimental.pallas import tpu as pltpu

_ROWS = 512
_VOCAB = 1000
_EMB = 64


def _argmax_embed_kernel(x_ref, dict_ref, out_ref):
    xb = x_ref[...]
    iota = lax.broadcasted_iota(jnp.int32, xb.shape, 1)
    m = jnp.max(xb, axis=-1, keepdims=True)
    eq = xb == m
    idx = jnp.min(jnp.where(eq, iota, _VOCAB), axis=-1, keepdims=True)
    onehot = (iota == idx).astype(jnp.float32)
    out_ref[...] = jnp.dot(onehot, dict_ref[...],
                           preferred_element_type=jnp.float32)


@jax.jit
def kernel(x, dictionary_weight):
    b, s, v = x.shape
    n = b * s
    x2 = x.reshape(n, v)
    grid = (n // _ROWS,)
    out = pl.pallas_call(
        _argmax_embed_kernel,
        grid=grid,
        in_specs=[
            pl.BlockSpec((_ROWS, v), lambda i: (i, 0)),
            pl.BlockSpec((_VOCAB, _EMB), lambda i: (0, 0)),
        ],
        out_specs=pl.BlockSpec((_ROWS, _EMB), lambda i: (i, 0)),
        out_shape=jax.ShapeDtypeStruct((n, _EMB), jnp.float32),
        compiler_params=pltpu.CompilerParams(
            dimension_semantics=("parallel",)),
    )(x2, dictionary_weight)
    return out.reshape(b, s, _EMB)


# speedup vs baseline: 1.1095x; 1.1089x over previous
import functools

import jax
import jax.numpy as jnp
from jax import lax
from jax.experimental import pallas as pl
from jax.experimental.pallas import tpu as pltpu

_ROWS = 2048
_VOCAB = 1000
_EMB = 64


def _argmax_embed_kernel(x_ref, dict_ref, out_ref):
    xb = x_ref[...]
    iota = lax.broadcasted_iota(jnp.int32, xb.shape, 1)
    m = jnp.max(xb, axis=-1, keepdims=True)
    eq = xb == m
    idx = jnp.min(jnp.where(eq, iota, _VOCAB), axis=-1, keepdims=True)
    onehot = (iota == idx).astype(jnp.float32)
    out_ref[...] = jnp.dot(onehot, dict_ref[...],
                           preferred_element_type=jnp.float32)


@jax.jit
def kernel(x, dictionary_weight):
    b, s, v = x.shape
    n = b * s
    x2 = x.reshape(n, v)
    grid = (n // _ROWS,)
    out = pl.pallas_call(
        _argmax_embed_kernel,
        grid=grid,
        in_specs=[
            pl.BlockSpec((_ROWS, v), lambda i: (i, 0)),
            pl.BlockSpec((_VOCAB, _EMB), lambda i: (0, 0)),
        ],
        out_specs=pl.BlockSpec((_ROWS, _EMB), lambda i: (i, 0)),
        out_shape=jax.ShapeDtypeStruct((n, _EMB), jnp.float32),
        compiler_params=pltpu.CompilerParams(
            dimension_semantics=("parallel",)),
    )(x2, dictionary_weight)
    return out.reshape(b, s, _EMB)


# speedup vs baseline: 1.1316x; 1.0199x over previous
import functools

import jax
import jax.numpy as jnp
from jax import lax
from jax.experimental import pallas as pl
from jax.experimental.pallas import tpu as pltpu

_ROWS = 4096
_VOCAB = 1000
_EMB = 64


def _argmax_embed_kernel(x_ref, dict_ref, out_ref):
    xb = x_ref[...]
    iota = lax.broadcasted_iota(jnp.int32, xb.shape, 1)
    m = jnp.max(xb, axis=-1, keepdims=True)
    eq = xb == m
    idx = jnp.min(jnp.where(eq, iota, _VOCAB), axis=-1, keepdims=True)
    onehot = (iota == idx).astype(jnp.float32)
    out_ref[...] = jnp.dot(onehot, dict_ref[...],
                           preferred_element_type=jnp.float32)


@jax.jit
def kernel(x, dictionary_weight):
    b, s, v = x.shape
    n = b * s
    x2 = x.reshape(n, v)
    grid = (n // _ROWS,)
    out = pl.pallas_call(
        _argmax_embed_kernel,
        grid=grid,
        in_specs=[
            pl.BlockSpec((_ROWS, v), lambda i: (i, 0)),
            pl.BlockSpec((_VOCAB, _EMB), lambda i: (0, 0)),
        ],
        out_specs=pl.BlockSpec((_ROWS, _EMB), lambda i: (i, 0)),
        out_shape=jax.ShapeDtypeStruct((n, _EMB), jnp.float32),
        compiler_params=pltpu.CompilerParams(
            dimension_semantics=("parallel",)),
    )(x2, dictionary_weight)
    return out.reshape(b, s, _EMB)


# speedup vs baseline: 1.4374x; 1.2702x over previous
import jax
import jax.numpy as jnp
from jax import lax
from jax.experimental import pallas as pl
from jax.experimental.pallas import tpu as pltpu

_B = 64
_VOCAB = 1000
_EMB = 64


def _argmax_embed_kernel(x_ref, dict_ref, out_ref):
    xb = x_ref[...]
    iota = lax.broadcasted_iota(jnp.int32, xb.shape, 2)
    m = jnp.max(xb, axis=-1, keepdims=True)
    eq = xb == m
    idx = jnp.min(jnp.where(eq, iota, _VOCAB), axis=-1, keepdims=True)
    onehot = (iota == idx).astype(jnp.float32)
    out_ref[...] = lax.dot_general(
        onehot, dict_ref[...],
        dimension_numbers=(((2,), (0,)), ((), ())),
        preferred_element_type=jnp.float32)


@jax.jit
def kernel(x, dictionary_weight):
    b, s, v = x.shape
    grid = (b // _B,)
    return pl.pallas_call(
        _argmax_embed_kernel,
        grid=grid,
        in_specs=[
            pl.BlockSpec((_B, s, v), lambda i: (i, 0, 0)),
            pl.BlockSpec((_VOCAB, _EMB), lambda i: (0, 0)),
        ],
        out_specs=pl.BlockSpec((_B, s, _EMB), lambda i: (i, 0, 0)),
        out_shape=jax.ShapeDtypeStruct((b, s, _EMB), jnp.float32),
        compiler_params=pltpu.CompilerParams(
            dimension_semantics=("parallel",)),
    )(x, dictionary_weight)


# speedup vs baseline: 1.4411x; 1.0026x over previous
import jax
import jax.numpy as jnp
from jax import lax
from jax.experimental import pallas as pl
from jax.experimental.pallas import tpu as pltpu

_B = 32
_VOCAB = 1000
_EMB = 64


def _argmax_embed_one(xb, dict_w):
    iota = lax.broadcasted_iota(jnp.int32, xb.shape, 2)
    m = jnp.max(xb, axis=-1, keepdims=True)
    eq = xb == m
    idx = jnp.min(jnp.where(eq, iota, _VOCAB), axis=-1, keepdims=True)
    onehot = (iota == idx).astype(jnp.float32)
    return lax.dot_general(
        onehot, dict_w,
        dimension_numbers=(((2,), (0,)), ((), ())),
        preferred_element_type=jnp.float32)


def _argmax_embed_kernel(xa_ref, xb_ref, dict_ref, out_ref):
    dict_w = dict_ref[...]
    out_ref[:_B] = _argmax_embed_one(xa_ref[...], dict_w)
    out_ref[_B:] = _argmax_embed_one(xb_ref[...], dict_w)


@jax.jit
def kernel(x, dictionary_weight):
    b, s, v = x.shape
    grid = (b // (2 * _B),)
    return pl.pallas_call(
        _argmax_embed_kernel,
        grid=grid,
        in_specs=[
            pl.BlockSpec((_B, s, v), lambda i: (2 * i, 0, 0)),
            pl.BlockSpec((_B, s, v), lambda i: (2 * i + 1, 0, 0)),
            pl.BlockSpec((_VOCAB, _EMB), lambda i: (0, 0)),
        ],
        out_specs=pl.BlockSpec((2 * _B, s, _EMB), lambda i: (i, 0, 0)),
        out_shape=jax.ShapeDtypeStruct((b, s, _EMB), jnp.float32),
        compiler_params=pltpu.CompilerParams(
            dimension_semantics=("parallel",)),
    )(x, x, dictionary_weight)
